# baseline (speedup 1.0000x reference)
import functools
import math

import jax
import jax.numpy as jnp
from jax import lax
from jax.experimental import pallas as pl


def _attn_kernel(x_ref, idx_ref, wgt_ref, ali_ref, dst_ref,
                 wq_ref, wk0_ref, wv0_ref, wx_ref, wp_ref, bp_ref,
                 out_ref, *, S, C, H, K):
    d = C // H
    f32 = jnp.float32
    xb = x_ref[0]
    q = jnp.dot(xb, wq_ref[...], preferred_element_type=f32, precision=lax.Precision.HIGHEST)
    xk = jnp.dot(xb, wk0_ref[...], preferred_element_type=f32, precision=lax.Precision.HIGHEST)
    xv = jnp.dot(xb, wv0_ref[...], preferred_element_type=f32, precision=lax.Precision.HIGHEST)
    xkv = jnp.concatenate([xk, xv], axis=1).astype(jnp.bfloat16)

    idxf = idx_ref[0, 0].astype(f32)
    alib = ali_ref[0, 0]
    dstb = dst_ref[0, 0]
    wgtb = wgt_ref[0, 0] + 1e-6

    iota_j = lax.broadcasted_iota(jnp.int32, (S, S), 1).astype(f32)
    e_blocks = []
    for k in range(K):
        sel = (lax.broadcasted_iota(jnp.int32, (K, S), 0) == k).astype(f32)
        idx_bc = jnp.dot(idxf, sel, preferred_element_type=f32, precision=lax.Precision.HIGHEST)
        e_blocks.append((idx_bc == iota_j).astype(jnp.bfloat16))
    E = jnp.concatenate(e_blocks, axis=0)
    G = jnp.dot(E, xkv, preferred_element_type=f32)

    bd = (lax.broadcasted_iota(jnp.int32, (C, H), 0) // d ==
          lax.broadcasted_iota(jnp.int32, (C, H), 1)).astype(f32)
    bdT = bd.T
    M = K * H
    repK = (lax.broadcasted_iota(jnp.int32, (K, M), 1) // H ==
            lax.broadcasted_iota(jnp.int32, (K, M), 0)).astype(f32)
    repH = (lax.broadcasted_iota(jnp.int32, (H, M), 1) % H ==
            lax.broadcasted_iota(jnp.int32, (H, M), 0)).astype(f32)
    redH = repH.T

    c1 = jnp.dot(q * wx_ref[0:1, :], bd, preferred_element_type=f32, precision=lax.Precision.HIGHEST)
    c2 = jnp.dot(q * wx_ref[1:2, :], bd, preferred_element_type=f32, precision=lax.Precision.HIGHEST)

    kq_blocks = [
        jnp.dot(G[k * S:(k + 1) * S, :C] * q, bd, preferred_element_type=f32, precision=lax.Precision.HIGHEST)
        for k in range(K)
    ]
    kq = jnp.concatenate(kq_blocks, axis=1)

    ali64 = jnp.dot(alib, repK, preferred_element_type=f32, precision=lax.Precision.HIGHEST)
    dst64 = jnp.dot(dstb, repK, preferred_element_type=f32, precision=lax.Precision.HIGHEST)
    wgt64 = jnp.dot(wgtb, repK, preferred_element_type=f32, precision=lax.Precision.HIGHEST)
    c164 = jnp.dot(c1, repH, preferred_element_type=f32, precision=lax.Precision.HIGHEST)
    c264 = jnp.dot(c2, repH, preferred_element_type=f32, precision=lax.Precision.HIGHEST)

    scale = 1.0 / math.sqrt(d)
    p64 = jnp.exp((kq + ali64 * c164 + dst64 * c264) * scale) * wgt64

    den = jnp.dot(p64, redH, preferred_element_type=f32, precision=lax.Precision.HIGHEST)
    pa = jnp.dot(p64 * ali64, redH, preferred_element_type=f32, precision=lax.Precision.HIGHEST)
    pd = jnp.dot(p64 * dst64, redH, preferred_element_type=f32, precision=lax.Precision.HIGHEST)

    num = jnp.zeros((S, C), f32)
    for k in range(K):
        p_exp = jnp.dot(p64[:, H * k:H * (k + 1)], bdT,
                        preferred_element_type=f32, precision=lax.Precision.HIGHEST)
        num = num + p_exp * G[k * S:(k + 1) * S, C:]
    num = num + jnp.dot(pa, bdT, preferred_element_type=f32, precision=lax.Precision.HIGHEST) * wx_ref[2:3, :]
    num = num + jnp.dot(pd, bdT, preferred_element_type=f32, precision=lax.Precision.HIGHEST) * wx_ref[3:4, :]

    out = num / jnp.dot(den, bdT, preferred_element_type=f32, precision=lax.Precision.HIGHEST)
    out = jnp.dot(out, wp_ref[...], preferred_element_type=f32, precision=lax.Precision.HIGHEST) + bp_ref[0:1, :]
    out_ref[0] = out


def kernel(x, spatial_idx, spatial_wgt, alignment, dist, Wq, Wk, Wv, Wp, bp):
    B, S, T, C = x.shape
    K = spatial_idx.shape[-1]
    H = 4
    f32 = jnp.float32

    idx = spatial_idx.astype(jnp.int32)

    wx = jnp.concatenate([Wk[C:C + 2], Wv[C:C + 2],
                          jnp.zeros((4, C), f32)], axis=0)
    bp_pad = jnp.concatenate([bp.reshape(1, C), jnp.zeros((7, C), f32)], axis=0)

    x2 = x.reshape(B, S, T * C)

    bspec_x = pl.BlockSpec((1, S, C), lambda b, t: (b, 0, t))
    bspec_sk = lambda: pl.BlockSpec((1, 1, S, K), lambda b, t: (b, t, 0, 0))
    bspec_w = lambda shape: pl.BlockSpec(shape, lambda b, t: (0, 0))

    out = pl.pallas_call(
        functools.partial(_attn_kernel, S=S, C=C, H=H, K=K),
        grid=(B, T),
        in_specs=[
            bspec_x,
            bspec_sk(),
            bspec_sk(),
            bspec_sk(),
            bspec_sk(),
            bspec_w((C, C)),
            bspec_w((C, C)),
            bspec_w((C, C)),
            bspec_w((8, C)),
            bspec_w((C, C)),
            bspec_w((8, C)),
        ],
        out_specs=bspec_x,
        out_shape=jax.ShapeDtypeStruct((B, S, T * C), f32),
    )(x2, idx, spatial_wgt, alignment, dist, Wq, Wk[:C], Wv[:C], wx, Wp, bp_pad)

    return out.reshape(B, S, T, C)


# speedup vs baseline: 2.3850x; 2.3850x over previous
import functools
import math

import jax
import jax.numpy as jnp
from jax import lax
from jax.experimental import pallas as pl


def _attn_kernel(x_ref, idx_ref, wgt_ref, ali_ref, dst_ref,
                 wq_ref, wk0_ref, wv0_ref, wx_ref, wp_ref, bp_ref,
                 iotaj_ref, repk_ref, reph_ref, bd_ref, bdt_ref, redh_ref,
                 out_ref, *, S, C, H, K):
    d = C // H
    f32 = jnp.float32
    i32 = jnp.int32
    NB = S // 128
    M = K * H
    xb = x_ref[0]
    q = jnp.dot(xb, wq_ref[...], preferred_element_type=f32)
    xk = jnp.dot(xb, wk0_ref[...], preferred_element_type=f32)
    xv = jnp.dot(xb, wv0_ref[...], preferred_element_type=f32)
    xv_bf = xv.astype(jnp.bfloat16)

    idxb = idx_ref[0, 0]
    alib = ali_ref[0, 0]
    dstb = dst_ref[0, 0]
    wgtb = wgt_ref[0, 0] + 1e-6

    iota_j = iotaj_ref[...]
    e_blocks = [(idxb[:, k:k + 1] == iota_j).astype(jnp.bfloat16)
                for k in range(K)]
    E = jnp.concatenate(e_blocks, axis=0)
    Gv = jnp.dot(E, xv_bf, preferred_element_type=f32)

    qk_heads = [
        lax.dot_general(q[:, h * d:(h + 1) * d], xk[:, h * d:(h + 1) * d],
                        ((( 1,), (1,)), ((), ())),
                        preferred_element_type=f32)
        for h in range(H)
    ]

    rep_k = repk_ref[...]
    rep_h = reph_ref[...]

    zpadK = jnp.zeros((S, 128 - 3 * K), f32)
    adw = jnp.concatenate([alib, dstb, wgtb, zpadK], axis=1)
    ali64 = jnp.take_along_axis(adw, rep_k, axis=1)
    dst64 = jnp.take_along_axis(adw, K + rep_k, axis=1)
    wgt64 = jnp.take_along_axis(adw, 2 * K + rep_k, axis=1)
    idxp = jnp.concatenate(
        [idxb, jnp.zeros((S, 128 - K), i32)], axis=1)
    idx64 = jnp.take_along_axis(idxp, rep_k, axis=1)
    lo64 = jnp.bitwise_and(idx64, 127)
    hi64 = jnp.right_shift(idx64, 7)

    bd = bd_ref[...]
    bdT = bdt_ref[...]
    c1 = jnp.dot(q * wx_ref[0:1, :], bd, preferred_element_type=f32)
    c2 = jnp.dot(q * wx_ref[1:2, :], bd, preferred_element_type=f32)
    cc = jnp.concatenate(
        [c1, c2, jnp.zeros((S, 128 - 2 * H), f32)], axis=1)
    c164 = jnp.take_along_axis(cc, rep_h, axis=1)
    c264 = jnp.take_along_axis(cc, H + rep_h, axis=1)

    kq64 = jnp.zeros((S, M), f32)
    for b in range(NB):
        mb = hi64 == b
        for h in range(H):
            g = jnp.take_along_axis(qk_heads[h][:, b * 128:(b + 1) * 128],
                                    lo64, axis=1)
            kq64 = kq64 + jnp.where(mb & (rep_h == h), g, 0.0)

    scale = 1.0 / math.sqrt(d)
    p64 = jnp.exp((kq64 + ali64 * c164 + dst64 * c264) * scale) * wgt64

    redH = redh_ref[...]
    den = jnp.dot(p64, redH, preferred_element_type=f32)
    pa = jnp.dot(p64 * ali64, redH, preferred_element_type=f32)
    pd = jnp.dot(p64 * dst64, redH, preferred_element_type=f32)


    terms = [
        jnp.dot(p64[:, H * k:H * (k + 1)], bdT,
                preferred_element_type=f32) * Gv[k * S:(k + 1) * S]
        for k in range(K)
    ]
    while len(terms) > 1:
        terms = [a + b for a, b in zip(terms[::2], terms[1::2])]
    num = terms[0]
    num = num + jnp.dot(pa, bdT, preferred_element_type=f32) * wx_ref[2:3, :]
    num = num + jnp.dot(pd, bdT, preferred_element_type=f32) * wx_ref[3:4, :]

    out = num / jnp.dot(den, bdT, preferred_element_type=f32)
    out = jnp.dot(out, wp_ref[...], preferred_element_type=f32) + bp_ref[0:1, :]
    out_ref[0] = out


def kernel(x, spatial_idx, spatial_wgt, alignment, dist, Wq, Wk, Wv, Wp, bp):
    B, S, T, C = x.shape
    K = spatial_idx.shape[-1]
    H = 4
    f32 = jnp.float32

    idx = spatial_idx.astype(jnp.int32)

    wx = jnp.concatenate([Wk[C:C + 2], Wv[C:C + 2],
                          jnp.zeros((4, C), f32)], axis=0)
    bp_pad = jnp.concatenate([bp.reshape(1, C), jnp.zeros((7, C), f32)], axis=0)

    x2 = x.reshape(B, S, T * C)

    d = C // H
    M = K * H
    i32 = jnp.int32
    iota_j = jax.lax.broadcasted_iota(i32, (S, S), 1)
    iota_m = jax.lax.broadcasted_iota(i32, (S, M), 1)
    rep_k = iota_m // H
    rep_h = iota_m % H
    bd = (jax.lax.broadcasted_iota(i32, (C, H), 0) // d ==
          jax.lax.broadcasted_iota(i32, (C, H), 1)).astype(f32)
    bdT = bd.T
    redH = (jax.lax.broadcasted_iota(i32, (M, H), 0) % H ==
            jax.lax.broadcasted_iota(i32, (M, H), 1)).astype(f32)

    bspec_x = pl.BlockSpec((1, S, C), lambda b, t: (b, 0, t))
    bspec_sk = lambda: pl.BlockSpec((1, 1, S, K), lambda b, t: (b, t, 0, 0))
    bspec_w = lambda shape: pl.BlockSpec(shape, lambda b, t: (0, 0))

    out = pl.pallas_call(
        functools.partial(_attn_kernel, S=S, C=C, H=H, K=K),
        grid=(B, T),
        in_specs=[
            bspec_x,
            bspec_sk(),
            bspec_sk(),
            bspec_sk(),
            bspec_sk(),
            bspec_w((C, C)),
            bspec_w((C, C)),
            bspec_w((C, C)),
            bspec_w((8, C)),
            bspec_w((C, C)),
            bspec_w((8, C)),
            bspec_w((S, S)),
            bspec_w((S, M)),
            bspec_w((S, M)),
            bspec_w((C, H)),
            bspec_w((H, C)),
            bspec_w((M, H)),
        ],
        out_specs=bspec_x,
        out_shape=jax.ShapeDtypeStruct((B, S, T * C), f32),
    )(x2, idx, spatial_wgt, alignment, dist, Wq, Wk[:C], Wv[:C], wx, Wp, bp_pad,
      iota_j, rep_k, rep_h, bd, bdT, redH)

    return out.reshape(B, S, T, C)


# speedup vs baseline: 2.8944x; 1.2136x over previous
import functools
import math

import jax
import jax.numpy as jnp
from jax import lax
from jax.experimental import pallas as pl


def _attn_kernel(x_ref, idx_ref, wgt_ref, ali_ref, dst_ref,
                 wq_ref, wk0_ref, wv0_ref, wx_ref, wp_ref, bp_ref,
                 out_ref, *, S, C, H, K):
    d = C // H
    f32 = jnp.float32
    bf16 = jnp.bfloat16
    i32 = jnp.int32
    M = K * H
    xb = x_ref[0]
    q = jnp.dot(xb, wq_ref[...], preferred_element_type=f32)
    xk = jnp.dot(xb, wk0_ref[...], preferred_element_type=f32).astype(bf16)
    xv = jnp.dot(xb, wv0_ref[...], preferred_element_type=f32).astype(bf16)
    xkv = jnp.concatenate([xk, xv], axis=1)

    idxb = idx_ref[0]
    alib = ali_ref[0]
    dstb = dst_ref[0]
    wgtb = wgt_ref[0] + 1e-6

    iota_j = lax.broadcasted_iota(i32, (S, S), 1)
    e_blocks = [(idxb[:, k:k + 1] == iota_j).astype(bf16) for k in range(K)]
    E = jnp.concatenate(e_blocks, axis=0)
    G = jnp.dot(E, xkv, preferred_element_type=f32).astype(bf16)

    bd = (lax.broadcasted_iota(i32, (C, H), 0) // d ==
          lax.broadcasted_iota(i32, (C, H), 1)).astype(f32)
    bdT = bd.T

    c1 = jnp.dot(q * wx_ref[0:1, :], bd, preferred_element_type=f32)
    c2 = jnp.dot(q * wx_ref[1:2, :], bd, preferred_element_type=f32)

    kq64 = jnp.concatenate([
        jnp.dot(G[k * S:(k + 1) * S, :C] * q, bd, preferred_element_type=f32)
        for k in range(K)
    ], axis=1)

    iota_m = lax.broadcasted_iota(i32, (S, M), 1)
    rep_k = iota_m // H
    rep_h = iota_m % H
    adw = jnp.concatenate(
        [alib, dstb, wgtb, jnp.zeros((S, 128 - 3 * K), f32)], axis=1)
    ali64 = jnp.take_along_axis(adw, rep_k, axis=1)
    dst64 = jnp.take_along_axis(adw, K + rep_k, axis=1)
    wgt64 = jnp.take_along_axis(adw, 2 * K + rep_k, axis=1)
    cc = jnp.concatenate(
        [c1, c2, jnp.zeros((S, 128 - 2 * H), f32)], axis=1)
    c164 = jnp.take_along_axis(cc, rep_h, axis=1)
    c264 = jnp.take_along_axis(cc, H + rep_h, axis=1)

    scale = 1.0 / math.sqrt(d)
    p64 = jnp.exp((kq64 + ali64 * c164 + dst64 * c264) * scale) * wgt64

    redH = (lax.broadcasted_iota(i32, (M, H), 0) % H ==
            lax.broadcasted_iota(i32, (M, H), 1)).astype(f32)
    den = jnp.dot(p64, redH, preferred_element_type=f32)
    pa = jnp.dot(p64 * ali64, redH, preferred_element_type=f32)
    pd = jnp.dot(p64 * dst64, redH, preferred_element_type=f32)

    terms = [
        jnp.dot(p64[:, H * k:H * (k + 1)], bdT,
                preferred_element_type=f32) * G[k * S:(k + 1) * S, C:]
        for k in range(K)
    ]
    while len(terms) > 1:
        terms = [a + b for a, b in zip(terms[::2], terms[1::2])]
    num = terms[0]
    num = num + jnp.dot(pa, bdT, preferred_element_type=f32) * wx_ref[2:3, :]
    num = num + jnp.dot(pd, bdT, preferred_element_type=f32) * wx_ref[3:4, :]

    out = num / jnp.dot(den, bdT, preferred_element_type=f32)
    out = jnp.dot(out, wp_ref[...], preferred_element_type=f32) + bp_ref[0:1, :]
    out_ref[0] = out


def kernel(x, spatial_idx, spatial_wgt, alignment, dist, Wq, Wk, Wv, Wp, bp):
    B, S, T, C = x.shape
    K = spatial_idx.shape[-1]
    H = 4
    BT = B * T
    f32 = jnp.float32

    x_ = jnp.transpose(x, (0, 2, 1, 3)).reshape(BT, S, C)
    idx = spatial_idx.reshape(BT, S, K).astype(jnp.int32)
    wgt = spatial_wgt.reshape(BT, S, K)
    ali = alignment.reshape(BT, S, K)
    dst = dist.reshape(BT, S, K)

    wx = jnp.concatenate([Wk[C:C + 2], Wv[C:C + 2],
                          jnp.zeros((4, C), f32)], axis=0)
    bp_pad = jnp.concatenate([bp.reshape(1, C), jnp.zeros((7, C), f32)], axis=0)

    grid = (BT,)
    bspec_bt = lambda: pl.BlockSpec((1, S, C), lambda i: (i, 0, 0))
    bspec_sk = lambda: pl.BlockSpec((1, S, K), lambda i: (i, 0, 0))
    bspec_w = lambda shape: pl.BlockSpec(shape, lambda i: (0, 0))

    out = pl.pallas_call(
        functools.partial(_attn_kernel, S=S, C=C, H=H, K=K),
        grid=grid,
        in_specs=[
            bspec_bt(),
            bspec_sk(),
            bspec_sk(),
            bspec_sk(),
            bspec_sk(),
            bspec_w((C, C)),
            bspec_w((C, C)),
            bspec_w((C, C)),
            bspec_w((8, C)),
            bspec_w((C, C)),
            bspec_w((8, C)),
        ],
        out_specs=bspec_bt(),
        out_shape=jax.ShapeDtypeStruct((BT, S, C), f32),
    )(x_, idx, wgt, ali, dst, Wq, Wk[:C], Wv[:C], wx, Wp, bp_pad)

    return out.reshape(B, T, S, C).transpose(0, 2, 1, 3)


# speedup vs baseline: 2.9431x; 1.0168x over previous
import functools
import math

import jax
import jax.numpy as jnp
from jax import lax
from jax.experimental import pallas as pl


def _attn_kernel(x_ref, idx_ref, wgt_ref, ali_ref, dst_ref,
                 wq_ref, wkv_ref, wx_ref, wp_ref, bp_ref,
                 out_ref, *, S, C, H, K):
    d = C // H
    f32 = jnp.float32
    bf16 = jnp.bfloat16
    i32 = jnp.int32
    M = K * H
    xb = x_ref[0]
    q = jnp.dot(xb, wq_ref[...], preferred_element_type=f32)
    xkv = jnp.dot(xb, wkv_ref[...],
                  preferred_element_type=f32).astype(bf16)

    idxb = idx_ref[0]
    alib = ali_ref[0]
    dstb = dst_ref[0]
    wgtb = wgt_ref[0] + 1e-6

    iota_j = lax.broadcasted_iota(i32, (S, S), 1)
    e_blocks = [(idxb[:, k:k + 1] == iota_j).astype(bf16) for k in range(K)]
    E = jnp.concatenate(e_blocks, axis=0)
    G = jnp.dot(E, xkv, preferred_element_type=f32).astype(bf16)

    bd = (lax.broadcasted_iota(i32, (C, H), 0) // d ==
          lax.broadcasted_iota(i32, (C, H), 1)).astype(f32)
    bdT = bd.T

    bdw = jnp.concatenate([bd * wx_ref[0:1, :].T, bd * wx_ref[1:2, :].T],
                          axis=1)
    cc12 = jnp.dot(q, bdw, preferred_element_type=f32)
    c1 = cc12[:, :H]
    c2 = cc12[:, H:]

    kq64 = jnp.concatenate([
        jnp.dot(G[k * S:(k + 1) * S, :C] * q, bd, preferred_element_type=f32)
        for k in range(K)
    ], axis=1)

    iota_m = lax.broadcasted_iota(i32, (S, M), 1)
    rep_k = iota_m // H
    rep_h = iota_m % H
    adw = jnp.concatenate(
        [alib, dstb, wgtb, jnp.zeros((S, 128 - 3 * K), f32)], axis=1)
    ali64 = jnp.take_along_axis(adw, rep_k, axis=1)
    dst64 = jnp.take_along_axis(adw, K + rep_k, axis=1)
    wgt64 = jnp.take_along_axis(adw, 2 * K + rep_k, axis=1)
    cc = jnp.concatenate(
        [c1, c2, jnp.zeros((S, 128 - 2 * H), f32)], axis=1)
    c164 = jnp.take_along_axis(cc, rep_h, axis=1)
    c264 = jnp.take_along_axis(cc, H + rep_h, axis=1)

    scale = 1.0 / math.sqrt(d)
    p64 = jnp.exp((kq64 + ali64 * c164 + dst64 * c264) * scale) * wgt64

    redH = (lax.broadcasted_iota(i32, (M, H), 0) % H ==
            lax.broadcasted_iota(i32, (M, H), 1)).astype(f32)
    den = jnp.dot(p64, redH, preferred_element_type=f32)
    pa = jnp.dot(p64 * ali64, redH, preferred_element_type=f32)
    pd = jnp.dot(p64 * dst64, redH, preferred_element_type=f32)

    terms = [
        jnp.dot(p64[:, H * k:H * (k + 1)], bdT,
                preferred_element_type=f32) * G[k * S:(k + 1) * S, C:]
        for k in range(K)
    ]
    while len(terms) > 1:
        terms = [a + b for a, b in zip(terms[::2], terms[1::2])]
    num = terms[0]
    num = num + jnp.dot(pa, bdT, preferred_element_type=f32) * wx_ref[2:3, :]
    num = num + jnp.dot(pd, bdT, preferred_element_type=f32) * wx_ref[3:4, :]

    out = num / jnp.dot(den, bdT, preferred_element_type=f32)
    out = jnp.dot(out, wp_ref[...], preferred_element_type=f32) + bp_ref[0:1, :]
    out_ref[0] = out


def kernel(x, spatial_idx, spatial_wgt, alignment, dist, Wq, Wk, Wv, Wp, bp):
    B, S, T, C = x.shape
    K = spatial_idx.shape[-1]
    H = 4
    BT = B * T
    f32 = jnp.float32

    x_ = jnp.transpose(x, (0, 2, 1, 3)).reshape(BT, S, C)
    idx = spatial_idx.reshape(BT, S, K).astype(jnp.int32)
    wgt = spatial_wgt.reshape(BT, S, K)
    ali = alignment.reshape(BT, S, K)
    dst = dist.reshape(BT, S, K)

    wx = jnp.concatenate([Wk[C:C + 2], Wv[C:C + 2],
                          jnp.zeros((4, C), f32)], axis=0)
    bp_pad = jnp.concatenate([bp.reshape(1, C), jnp.zeros((7, C), f32)], axis=0)

    grid = (BT,)
    bspec_bt = lambda: pl.BlockSpec((1, S, C), lambda i: (i, 0, 0))
    bspec_sk = lambda: pl.BlockSpec((1, S, K), lambda i: (i, 0, 0))
    bspec_w = lambda shape: pl.BlockSpec(shape, lambda i: (0, 0))

    out = pl.pallas_call(
        functools.partial(_attn_kernel, S=S, C=C, H=H, K=K),
        grid=grid,
        in_specs=[
            bspec_bt(),
            bspec_sk(),
            bspec_sk(),
            bspec_sk(),
            bspec_sk(),
            bspec_w((C, C)),
            bspec_w((C, 2 * C)),
            bspec_w((8, C)),
            bspec_w((C, C)),
            bspec_w((8, C)),
        ],
        out_specs=bspec_bt(),
        out_shape=jax.ShapeDtypeStruct((BT, S, C), f32),
    )(x_, idx, wgt, ali, dst, Wq, jnp.concatenate([Wk[:C], Wv[:C]], axis=1),
      wx, Wp, bp_pad)

    return out.reshape(B, T, S, C).transpose(0, 2, 1, 3)
